# Initial kernel scaffold; baseline (speedup 1.0000x reference)
import functools

import jax
import jax.numpy as jnp
from jax import lax
from jax.experimental import pallas as pl
from jax.experimental.pallas import tpu as pltpu
from jax.experimental.pallas import tpu_sc as plsc

F32 = jnp.float32
I32 = jnp.int32

N_B, E_B, D_B = 50000, 800000, 64
N_C, E_C, D_C = 10000, 160000, 128
H, OUT = 64, 2

NCORE, NSUB = 2, 16
CH = 128
BLK = 512

N_BP = 50176
N_CP = 10240
E_BP = 6272 * CH
E_CP = 1280 * CH
NMAP = 416 * CH

_MESH = plsc.VectorSubcoreMesh(core_axis_name="c", subcore_axis_name="s",
                               num_cores=NCORE, num_subcores=NSUB)


def _fill_const(buf, nrows, ncol16, val):
    def row(i, _):
        for k in range(ncol16):
            buf[i, pl.ds(k * 16, 16)] = jnp.full((16,), val, F32)
        return 0
    lax.fori_loop(0, nrows, row, 0)


@functools.partial(
    pl.kernel,
    out_type=(jax.ShapeDtypeStruct((N_BP, 16), F32),
              jax.ShapeDtypeStruct((N_CP, 16), F32)),
    mesh=_MESH,
    scratch_types=[
        pltpu.VMEM((CH, 16), F32),
        pltpu.VMEM((16, CH), I32),
        pltpu.VMEM_SHARED((N_BP, 16), F32),
        pltpu.VMEM_SHARED((N_CP, 16), F32),
        pltpu.SemaphoreType.DMA,
    ],
)
def _sc_degrees(bdst_ref, cdst_ref, degb_ref, degc_ref,
                ones_v, idx_v, accb, accc, sem):
    c = lax.axis_index("c")
    s = lax.axis_index("s")
    _fill_const(ones_v, CH, 1, 1.0)

    def init(acc, rows_per_tile):
        base = s * rows_per_tile
        nfull, rem = divmod(rows_per_tile, CH)

        def blk(t, _):
            pltpu.sync_copy(ones_v, acc.at[pl.ds(base + t * CH, CH)])
            return 0
        lax.fori_loop(0, nfull, blk, 0)
        if rem:
            pltpu.sync_copy(ones_v.at[pl.ds(0, rem)],
                            acc.at[pl.ds(base + nfull * CH, rem)])

    def count(acc, dst_ref, nchunks, bsz):
        per_tile = nchunks // NSUB
        nb = per_tile // bsz

        def batch(b, _):
            cb = s * per_tile + b * bsz
            pltpu.sync_copy(dst_ref.at[pl.ds(cb, bsz)], idx_v.at[pl.ds(0, bsz)])
            descs = [pltpu.async_copy(ones_v, acc.at[idx_v.at[j]], sem,
                                      add=True)
                     for j in range(bsz)]
            for d in descs:
                d.wait()
            return 0
        lax.fori_loop(0, nb, batch, 0)

    @pl.when(c == 0)
    def _():
        init(accb, N_BP // NSUB)

    @pl.when(c == 1)
    def _():
        init(accc, N_CP // NSUB)

    plsc.subcore_barrier()

    @pl.when(c == 0)
    def _():
        count(accb, bdst_ref, E_BP // CH, 8)

    @pl.when(c == 1)
    def _():
        count(accc, cdst_ref, E_CP // CH, 8)

    plsc.subcore_barrier()

    @pl.when(c == 0)
    def _():
        r = N_BP // NSUB
        pltpu.sync_copy(accb.at[pl.ds(s * r, r)], degb_ref.at[pl.ds(s * r, r)])

    @pl.when(c == 1)
    def _():
        r = N_CP // NSUB
        pltpu.sync_copy(accc.at[pl.ds(s * r, r)], degc_ref.at[pl.ds(s * r, r)])


def _make_propagate(nrows, nchunks, bsz, fdim, feature_split):
    per_tile = nchunks // NSUB if feature_split else nchunks // (NSUB * NCORE)
    nb = per_tile // bsz
    assert per_tile % bsz == 0
    rpt = nrows // NSUB
    nfull, rem = divmod(rpt, CH)

    def body(y0, y1, src_ref, dst_ref, o0, o1,
             zero_v, sidx, didx, rows_v, acc, gsem, ssem):
        c = lax.axis_index("c")
        s = lax.axis_index("s")
        _fill_const(zero_v, CH, fdim // 16, 0.0)
        base_r = s * rpt

        def zblk(t, _):
            pltpu.sync_copy(zero_v, acc.at[pl.ds(base_r + t * CH, CH)])
            return 0
        lax.fori_loop(0, nfull, zblk, 0)
        if rem:
            pltpu.sync_copy(zero_v.at[pl.ds(0, rem)],
                            acc.at[pl.ds(base_r + nfull * CH, rem)])
        plsc.subcore_barrier()

        def run(tbl):
            if feature_split:
                chunk0 = s * per_tile
            else:
                chunk0 = c * (nchunks // NCORE) + s * per_tile

            def batch(b, _):
                cb = chunk0 + b * bsz
                pltpu.sync_copy(src_ref.at[pl.ds(cb, bsz)],
                                sidx.at[pl.ds(0, bsz)])
                pltpu.sync_copy(dst_ref.at[pl.ds(cb, bsz)],
                                didx.at[pl.ds(0, bsz)])
                g = [pltpu.async_copy(tbl.at[sidx.at[j]],
                                      rows_v.at[pl.ds(j * CH, CH)], gsem)
                     for j in range(bsz)]
                for d in g:
                    d.wait()
                sc_ = [pltpu.async_copy(rows_v.at[pl.ds(j * CH, CH)],
                                        acc.at[didx.at[j]], ssem, add=True)
                       for j in range(bsz)]
                for d in sc_:
                    d.wait()
                return 0
            lax.fori_loop(0, nb, batch, 0)

        if feature_split:
            @pl.when(c == 0)
            def _():
                run(y0)

            @pl.when(c == 1)
            def _():
                run(y1)
        else:
            run(y0)
        plsc.subcore_barrier()

        @pl.when(c == 0)
        def _():
            pltpu.sync_copy(acc.at[pl.ds(base_r, rpt)],
                            o0.at[pl.ds(base_r, rpt)])

        @pl.when(c == 1)
        def _():
            pltpu.sync_copy(acc.at[pl.ds(base_r, rpt)],
                            o1.at[pl.ds(base_r, rpt)])

    return pl.kernel(
        body,
        out_type=(jax.ShapeDtypeStruct((nrows, fdim), F32),
                  jax.ShapeDtypeStruct((nrows, fdim), F32)),
        mesh=_MESH,
        scratch_types=[
            pltpu.VMEM((CH, fdim), F32),
            pltpu.VMEM((16, CH), I32),
            pltpu.VMEM((16, CH), I32),
            pltpu.VMEM((bsz * CH, fdim), F32),
            pltpu.VMEM_SHARED((nrows, fdim), F32),
            pltpu.SemaphoreType.DMA,
            pltpu.SemaphoreType.DMA,
        ],
    )


_prop_comm = _make_propagate(N_CP, E_CP // CH, 8, 32, True)
_prop_bldg = _make_propagate(N_BP, E_BP // CH, 8, 32, True)
_prop_out = _make_propagate(N_BP, E_BP // CH, 7, 16, False)


@functools.partial(
    pl.kernel,
    out_type=jax.ShapeDtypeStruct((NMAP, D_B), F32),
    mesh=_MESH,
    scratch_types=[
        pltpu.VMEM((13, CH), I32),
        pltpu.VMEM((13 * CH, D_B), F32),
        pltpu.SemaphoreType.DMA,
    ],
)
def _sc_gather_comm(tbl_ref, map_ref, out_ref, idx_v, rows_v, sem):
    c = lax.axis_index("c")
    s = lax.axis_index("s")
    w = s * NCORE + c
    pltpu.sync_copy(map_ref.at[pl.ds(w * 13, 13)], idx_v)
    g = [pltpu.async_copy(tbl_ref.at[idx_v.at[j]],
                          rows_v.at[pl.ds(j * CH, CH)], sem)
         for j in range(13)]
    for d in g:
        d.wait()
    pltpu.sync_copy(rows_v, out_ref.at[pl.ds(w * 13 * CH, 13 * CH)])


def _row_spec(cols):
    return pl.BlockSpec((BLK, cols), lambda i: (i, 0))


def _full_spec(shape):
    return pl.BlockSpec(shape, lambda i: (0, 0))


def _tc_comm1_body(cf, w, deg, ylo, yhi):
    dinv = lax.rsqrt(deg[...][:, 0:1])
    y = jnp.dot(cf[...], w[...], preferred_element_type=F32) * dinv
    ylo[...] = y[:, :32]
    yhi[...] = y[:, 32:]


def _tc_comm1(cf, w_c1, deg_c):
    return pl.pallas_call(
        _tc_comm1_body,
        grid=(N_CP // BLK,),
        in_specs=[_row_spec(D_C), _full_spec((D_C, H)), _row_spec(16)],
        out_specs=[_row_spec(32), _row_spec(32)],
        out_shape=[jax.ShapeDtypeStruct((N_CP, 32), F32)] * 2,
    )(cf, w_c1, deg_c)


def _halves(alo, ahi, ylo, yhi, deg, bv):
    dinv = lax.rsqrt(deg[...][:, 0:1])
    hlo = jnp.maximum(dinv * (alo[...] + ylo[...]) + bv[:, :32], 0.0)
    hhi = jnp.maximum(dinv * (ahi[...] + yhi[...]) + bv[:, 32:], 0.0)
    return dinv, hlo, hhi


def _tc_step_body(alo, ahi, ylo, yhi, deg, w, b, olo, ohi):
    dinv, hlo, hhi = _halves(alo, ahi, ylo, yhi, deg, b[...])
    wv = w[...]
    t = (jnp.dot(hlo, wv[:32], preferred_element_type=F32)
         + jnp.dot(hhi, wv[32:], preferred_element_type=F32))
    y = t * dinv
    olo[...] = y[:, :32]
    ohi[...] = y[:, 32:]


def _tc_step(alo, ahi, ylo, yhi, deg, w, b, npad):
    return pl.pallas_call(
        _tc_step_body,
        grid=(npad // BLK,),
        in_specs=[_row_spec(32)] * 4 + [_row_spec(16), _full_spec((H, H)),
                                        _full_spec((1, H))],
        out_specs=[_row_spec(32), _row_spec(32)],
        out_shape=[jax.ShapeDtypeStruct((npad, 32), F32)] * 2,
    )(alo, ahi, ylo, yhi, deg, w, b.reshape(1, H))


def _tc_comm_fin_body(alo, ahi, ylo, yhi, deg, b, out):
    _, hlo, hhi = _halves(alo, ahi, ylo, yhi, deg, b[...])
    out[...] = jnp.concatenate([hlo, hhi], axis=1)


def _tc_comm_fin(alo, ahi, ylo, yhi, deg, b):
    return pl.pallas_call(
        _tc_comm_fin_body,
        grid=(N_CP // BLK,),
        in_specs=[_row_spec(32)] * 4 + [_row_spec(16), _full_spec((1, H))],
        out_specs=_row_spec(H),
        out_shape=jax.ShapeDtypeStruct((N_C, H), F32),
    )(alo, ahi, ylo, yhi, deg, b.reshape(1, H))


def _tc_att_body(bf, bc, deg, wa, w1, ba, olo, ohi):
    dinv = lax.rsqrt(deg[...][:, 0:1])
    bfv, bcv = bf[...], bc[...]
    wav, w1v = wa[...], w1[...]
    z = (jnp.dot(bfv, wav[:D_B], preferred_element_type=F32)
         + jnp.dot(bcv, wav[D_B:], preferred_element_type=F32) + ba[...])
    m = jnp.max(z, axis=1, keepdims=True)
    e = jnp.exp(z - m)
    a = e / jnp.sum(e, axis=1, keepdims=True)
    t = (a[:, 0:1] * jnp.dot(bfv, w1v[:D_B], preferred_element_type=F32)
         + a[:, 1:2] * jnp.dot(bcv, w1v[D_B:], preferred_element_type=F32))
    y = t * dinv
    olo[...] = y[:, :32]
    ohi[...] = y[:, 32:]


def _tc_att(bf, bc, deg_b, w_att, b_att, w_b1):
    return pl.pallas_call(
        _tc_att_body,
        grid=(N_BP // BLK,),
        in_specs=[_row_spec(D_B), _row_spec(D_B), _row_spec(16),
                  _full_spec((D_B + H, OUT)), _full_spec((D_B + H, H)),
                  _full_spec((1, OUT))],
        out_specs=[_row_spec(32), _row_spec(32)],
        out_shape=[jax.ShapeDtypeStruct((N_BP, 32), F32)] * 2,
    )(bf, bc, deg_b, w_att, w_b1, b_att.reshape(1, OUT))


def _tc_mm3_body(alo, ahi, ylo, yhi, deg, b, w3, out):
    dinv, hlo, hhi = _halves(alo, ahi, ylo, yhi, deg, b[...])
    w3v = w3[...]
    t = (jnp.dot(hlo, w3v[:32], preferred_element_type=F32)
         + jnp.dot(hhi, w3v[32:], preferred_element_type=F32))
    y3 = t * dinv
    out[...] = jnp.concatenate([y3, jnp.zeros((BLK, 16 - OUT), F32)], axis=1)


def _tc_mm3(alo, ahi, ylo, yhi, deg, b, w3):
    return pl.pallas_call(
        _tc_mm3_body,
        grid=(N_BP // BLK,),
        in_specs=[_row_spec(32)] * 4 + [_row_spec(16), _full_spec((1, H)),
                                        _full_spec((H, OUT))],
        out_specs=_row_spec(16),
        out_shape=jax.ShapeDtypeStruct((N_BP, 16), F32),
    )(alo, ahi, ylo, yhi, deg, b.reshape(1, H), w3)


def _tc_final_body(p0, p1, y3, deg, b3, out):
    dinv = lax.rsqrt(deg[...][:, 0:1])
    x = dinv * (p0[...] + p1[...] + y3[...]) + b3[...]
    x2 = x[:, 0:OUT]
    m = jnp.max(x2, axis=1, keepdims=True)
    lse = m + jnp.log(jnp.sum(jnp.exp(x2 - m), axis=1, keepdims=True))
    out[...] = x2 - lse


def _tc_final(p0, p1, y3, deg_b, b3):
    b3p = jnp.concatenate([b3, jnp.zeros((16 - OUT,), F32)]).reshape(1, 16)
    return pl.pallas_call(
        _tc_final_body,
        grid=(N_BP // BLK,),
        in_specs=[_row_spec(16)] * 3 + [_row_spec(16), _full_spec((1, 16))],
        out_specs=_row_spec(OUT),
        out_shape=jax.ShapeDtypeStruct((N_B, OUT), F32),
    )(p0, p1, y3, deg_b, b3p)


def kernel(building_features, building_edge_index, community_features,
           community_edge_index, building_to_comm_mapping, W_c1, b_c1,
           W_c2, b_c2, W_att, b_att, W_b1, b_b1, W_b2, b_b2, W_b3, b_b3):
    bsrc, bdst = building_edge_index[0], building_edge_index[1]
    csrc, cdst = community_edge_index[0], community_edge_index[1]
    padb, padc = E_BP - E_B, E_CP - E_C
    bsrc2 = jnp.concatenate([bsrc, jnp.zeros((padb,), I32)]).reshape(-1, CH)
    bdst2 = jnp.concatenate([bdst, jnp.full((padb,), N_B, I32)]).reshape(-1, CH)
    csrc2 = jnp.concatenate([csrc, jnp.zeros((padc,), I32)]).reshape(-1, CH)
    cdst2 = jnp.concatenate([cdst, jnp.full((padc,), N_C, I32)]).reshape(-1, CH)
    map2 = jnp.concatenate(
        [building_to_comm_mapping,
         jnp.zeros((NMAP - N_B,), I32)]).reshape(-1, CH)

    deg_b, deg_c = _sc_degrees(bdst2, cdst2)

    y1lo, y1hi = _tc_comm1(community_features, W_c1, deg_c)
    a1lo, a1hi = _prop_comm(y1lo, y1hi, csrc2, cdst2)
    y2lo, y2hi = _tc_step(a1lo, a1hi, y1lo, y1hi, deg_c, W_c2, b_c1, N_CP)
    a2lo, a2hi = _prop_comm(y2lo, y2hi, csrc2, cdst2)
    comm_x = _tc_comm_fin(a2lo, a2hi, y2lo, y2hi, deg_c, b_c2)

    bc = _sc_gather_comm(comm_x, map2)
    yb1lo, yb1hi = _tc_att(building_features, bc, deg_b, W_att, b_att, W_b1)

    ab1lo, ab1hi = _prop_bldg(yb1lo, yb1hi, bsrc2, bdst2)
    yb2lo, yb2hi = _tc_step(ab1lo, ab1hi, yb1lo, yb1hi, deg_b, W_b2, b_b1,
                            N_BP)
    ab2lo, ab2hi = _prop_bldg(yb2lo, yb2hi, bsrc2, bdst2)
    y3 = _tc_mm3(ab2lo, ab2hi, yb2lo, yb2hi, deg_b, b_b2, W_b3)
    p0, p1 = _prop_out(y3, y3, bsrc2, bdst2)
    return _tc_final(p0, p1, y3, deg_b, b_b3)


# speedup vs baseline: 17.5176x; 17.5176x over previous
import functools

import jax
import jax.numpy as jnp
from jax import lax
from jax.experimental import pallas as pl
from jax.experimental.pallas import tpu as pltpu
from jax.experimental.pallas import tpu_sc as plsc

F32 = jnp.float32
I32 = jnp.int32

N_B, E_B, D_B = 50000, 800000, 64
N_C, E_C, D_C = 10000, 160000, 128
H, OUT = 64, 2

NCORE, NSUB = 2, 16
CH = 128
BLK = 512

N_BP = 50176
N_CP = 10240
E_BP = 6272 * CH
E_CP = 1280 * CH
NMAP = 416 * CH

_MESH = plsc.VectorSubcoreMesh(core_axis_name="c", subcore_axis_name="s",
                               num_cores=NCORE, num_subcores=NSUB)
_SC_PARAMS = pltpu.CompilerParams(use_tc_tiling_on_sc=False)


def _fill_const(buf, nrows, ncol16, val):
    def row(i, _):
        for k in range(ncol16):
            buf[i, pl.ds(k * 16, 16)] = jnp.full((16,), val, F32)
        return 0
    lax.fori_loop(0, nrows, row, 0)


@functools.partial(
    pl.kernel,
    out_type=(jax.ShapeDtypeStruct((N_BP, 16), F32),
              jax.ShapeDtypeStruct((N_CP, 16), F32)),
    mesh=_MESH,
    scratch_types=[
        pltpu.VMEM((CH, 16), F32),
        pltpu.VMEM((8, CH), I32),
        pltpu.VMEM_SHARED((N_BP, 16), F32),
        pltpu.VMEM_SHARED((N_CP, 16), F32),
        pltpu.SemaphoreType.DMA,
    ],
    compiler_params=_SC_PARAMS,
)
def _sc_degrees(bdst_ref, cdst_ref, degb_ref, degc_ref,
                ones_v, idx_v, accb, accc, sem):
    c = lax.axis_index("c")
    s = lax.axis_index("s")
    _fill_const(ones_v, CH, 1, 1.0)

    def init(acc, rows_per_tile):
        base = s * rows_per_tile
        nfull, rem = divmod(rows_per_tile, CH)

        def blk(t, _):
            pltpu.sync_copy(ones_v, acc.at[pl.ds(base + t * CH, CH)])
            return 0
        lax.fori_loop(0, nfull, blk, 0)
        if rem:
            pltpu.sync_copy(ones_v.at[pl.ds(0, rem)],
                            acc.at[pl.ds(base + nfull * CH, rem)])

    def count(acc, dst_ref, nchunks, bsz):
        per_tile = nchunks // NSUB
        nb = per_tile // bsz

        def batch(b, _):
            cb = s * per_tile + b * bsz
            pltpu.sync_copy(dst_ref.at[pl.ds(cb, bsz)], idx_v.at[pl.ds(0, bsz)])
            descs = [pltpu.async_copy(ones_v, acc.at[idx_v.at[j]], sem,
                                      add=True)
                     for j in range(bsz)]
            for d in descs:
                d.wait()
            return 0
        lax.fori_loop(0, nb, batch, 0)

    @pl.when(c == 0)
    def _():
        init(accb, N_BP // NSUB)

    @pl.when(c == 1)
    def _():
        init(accc, N_CP // NSUB)

    plsc.subcore_barrier()

    @pl.when(c == 0)
    def _():
        count(accb, bdst_ref, E_BP // CH, 8)

    @pl.when(c == 1)
    def _():
        count(accc, cdst_ref, E_CP // CH, 8)

    plsc.subcore_barrier()

    @pl.when(c == 0)
    def _():
        r = N_BP // NSUB
        pltpu.sync_copy(accb.at[pl.ds(s * r, r)], degb_ref.at[pl.ds(s * r, r)])

    @pl.when(c == 1)
    def _():
        r = N_CP // NSUB
        pltpu.sync_copy(accc.at[pl.ds(s * r, r)], degc_ref.at[pl.ds(s * r, r)])


def _make_propagate(nrows, nchunks, bsz, fdim, feature_split):
    per_tile = nchunks // NSUB if feature_split else nchunks // (NSUB * NCORE)
    nb = per_tile // bsz
    assert per_tile % bsz == 0
    rpt = nrows // NSUB
    ZR = CH
    nfull, rem = divmod(rpt, ZR)

    def body(y0, y1, src_ref, dst_ref, o0, o1,
             zero_v, sidx, didx, rows_v, acc, gsem, ssem):
        c = lax.axis_index("c")
        s = lax.axis_index("s")
        _fill_const(zero_v, ZR, fdim // 16, 0.0)
        base_r = s * rpt

        def zblk(t, _):
            pltpu.sync_copy(zero_v, acc.at[pl.ds(base_r + t * ZR, ZR)])
            return 0
        lax.fori_loop(0, nfull, zblk, 0)
        if rem:
            pltpu.sync_copy(zero_v.at[pl.ds(0, rem)],
                            acc.at[pl.ds(base_r + nfull * ZR, rem)])
        plsc.subcore_barrier()

        def run(tbl):
            if feature_split:
                chunk0 = s * per_tile
            else:
                chunk0 = c * (nchunks // NCORE) + s * per_tile

            def batch(b, _):
                cb = chunk0 + b * bsz
                pltpu.sync_copy(src_ref.at[pl.ds(cb, bsz)], sidx)
                pltpu.sync_copy(dst_ref.at[pl.ds(cb, bsz)], didx)
                g = [pltpu.async_copy(tbl.at[sidx.at[j]],
                                      rows_v.at[pl.ds(j * CH, CH)], gsem)
                     for j in range(bsz)]
                for d in g:
                    d.wait()
                sc_ = [pltpu.async_copy(rows_v.at[pl.ds(j * CH, CH)],
                                        acc.at[didx.at[j]], ssem, add=True)
                       for j in range(bsz)]
                for d in sc_:
                    d.wait()
                return 0
            lax.fori_loop(0, nb, batch, 0)

        if feature_split:
            @pl.when(c == 0)
            def _():
                run(y0)

            @pl.when(c == 1)
            def _():
                run(y1)
        else:
            run(y0)
        plsc.subcore_barrier()

        @pl.when(c == 0)
        def _():
            pltpu.sync_copy(acc.at[pl.ds(base_r, rpt)],
                            o0.at[pl.ds(base_r, rpt)])

        @pl.when(c == 1)
        def _():
            pltpu.sync_copy(acc.at[pl.ds(base_r, rpt)],
                            o1.at[pl.ds(base_r, rpt)])

    return pl.kernel(
        body,
        out_type=(jax.ShapeDtypeStruct((nrows, fdim), F32),
                  jax.ShapeDtypeStruct((nrows, fdim), F32)),
        mesh=_MESH,
        scratch_types=[
            pltpu.VMEM((ZR, fdim), F32),
            pltpu.VMEM((bsz, CH), I32),
            pltpu.VMEM((bsz, CH), I32),
            pltpu.VMEM((bsz * CH, fdim), F32),
            pltpu.VMEM_SHARED((nrows, fdim), F32),
            pltpu.SemaphoreType.DMA,
            pltpu.SemaphoreType.DMA,
        ],
        compiler_params=_SC_PARAMS,
    )


_prop_comm = _make_propagate(N_CP, E_CP // CH, 16, 32, True)
_prop_bldg = _make_propagate(N_BP, E_BP // CH, 4, 32, True)
_prop_out = _make_propagate(N_BP, E_BP // CH, 7, 16, False)


@functools.partial(
    pl.kernel,
    out_type=jax.ShapeDtypeStruct((NMAP, D_B), F32),
    mesh=_MESH,
    scratch_types=[
        pltpu.VMEM((13, CH), I32),
        pltpu.VMEM((13 * CH, D_B), F32),
        pltpu.SemaphoreType.DMA,
    ],
    compiler_params=_SC_PARAMS,
)
def _sc_gather_comm(tbl_ref, map_ref, out_ref, idx_v, rows_v, sem):
    c = lax.axis_index("c")
    s = lax.axis_index("s")
    w = s * NCORE + c
    pltpu.sync_copy(map_ref.at[pl.ds(w * 13, 13)], idx_v)
    g = [pltpu.async_copy(tbl_ref.at[idx_v.at[j]],
                          rows_v.at[pl.ds(j * CH, CH)], sem)
         for j in range(13)]
    for d in g:
        d.wait()
    pltpu.sync_copy(rows_v, out_ref.at[pl.ds(w * 13 * CH, 13 * CH)])


def _row_spec(cols):
    return pl.BlockSpec((BLK, cols), lambda i: (i, 0))


def _full_spec(shape):
    return pl.BlockSpec(shape, lambda i: (0, 0))


def _tc_comm1_body(cf, w, deg, ylo, yhi):
    dinv = lax.rsqrt(deg[...][:, 0:1])
    y = jnp.dot(cf[...], w[...], preferred_element_type=F32) * dinv
    ylo[...] = y[:, :32]
    yhi[...] = y[:, 32:]


def _tc_comm1(cf, w_c1, deg_c):
    return pl.pallas_call(
        _tc_comm1_body,
        grid=(N_CP // BLK,),
        in_specs=[_row_spec(D_C), _full_spec((D_C, H)), _row_spec(16)],
        out_specs=[_row_spec(32), _row_spec(32)],
        out_shape=[jax.ShapeDtypeStruct((N_CP, 32), F32)] * 2,
    )(cf, w_c1, deg_c)


def _halves(alo, ahi, ylo, yhi, deg, bv):
    dinv = lax.rsqrt(deg[...][:, 0:1])
    hlo = jnp.maximum(dinv * (alo[...] + ylo[...]) + bv[:, :32], 0.0)
    hhi = jnp.maximum(dinv * (ahi[...] + yhi[...]) + bv[:, 32:], 0.0)
    return dinv, hlo, hhi


def _tc_step_body(alo, ahi, ylo, yhi, deg, w, b, olo, ohi):
    dinv, hlo, hhi = _halves(alo, ahi, ylo, yhi, deg, b[...])
    wv = w[...]
    t = (jnp.dot(hlo, wv[:32], preferred_element_type=F32)
         + jnp.dot(hhi, wv[32:], preferred_element_type=F32))
    y = t * dinv
    olo[...] = y[:, :32]
    ohi[...] = y[:, 32:]


def _tc_step(alo, ahi, ylo, yhi, deg, w, b, npad):
    return pl.pallas_call(
        _tc_step_body,
        grid=(npad // BLK,),
        in_specs=[_row_spec(32)] * 4 + [_row_spec(16), _full_spec((H, H)),
                                        _full_spec((1, H))],
        out_specs=[_row_spec(32), _row_spec(32)],
        out_shape=[jax.ShapeDtypeStruct((npad, 32), F32)] * 2,
    )(alo, ahi, ylo, yhi, deg, w, b.reshape(1, H))


def _tc_comm_fin_body(alo, ahi, ylo, yhi, deg, b, out):
    _, hlo, hhi = _halves(alo, ahi, ylo, yhi, deg, b[...])
    out[...] = jnp.concatenate([hlo, hhi], axis=1)


def _tc_comm_fin(alo, ahi, ylo, yhi, deg, b):
    return pl.pallas_call(
        _tc_comm_fin_body,
        grid=(N_CP // BLK,),
        in_specs=[_row_spec(32)] * 4 + [_row_spec(16), _full_spec((1, H))],
        out_specs=_row_spec(H),
        out_shape=jax.ShapeDtypeStruct((N_C, H), F32),
    )(alo, ahi, ylo, yhi, deg, b.reshape(1, H))


def _tc_att_body(bf, bc, deg, wa, w1, ba, olo, ohi):
    dinv = lax.rsqrt(deg[...][:, 0:1])
    bfv, bcv = bf[...], bc[...]
    wav, w1v = wa[...], w1[...]
    z = (jnp.dot(bfv, wav[:D_B], preferred_element_type=F32)
         + jnp.dot(bcv, wav[D_B:], preferred_element_type=F32) + ba[...])
    m = jnp.max(z, axis=1, keepdims=True)
    e = jnp.exp(z - m)
    a = e / jnp.sum(e, axis=1, keepdims=True)
    t = (a[:, 0:1] * jnp.dot(bfv, w1v[:D_B], preferred_element_type=F32)
         + a[:, 1:2] * jnp.dot(bcv, w1v[D_B:], preferred_element_type=F32))
    y = t * dinv
    olo[...] = y[:, :32]
    ohi[...] = y[:, 32:]


def _tc_att(bf, bc, deg_b, w_att, b_att, w_b1):
    return pl.pallas_call(
        _tc_att_body,
        grid=(N_BP // BLK,),
        in_specs=[_row_spec(D_B), _row_spec(D_B), _row_spec(16),
                  _full_spec((D_B + H, OUT)), _full_spec((D_B + H, H)),
                  _full_spec((1, OUT))],
        out_specs=[_row_spec(32), _row_spec(32)],
        out_shape=[jax.ShapeDtypeStruct((N_BP, 32), F32)] * 2,
    )(bf, bc, deg_b, w_att, w_b1, b_att.reshape(1, OUT))


def _tc_mm3_body(alo, ahi, ylo, yhi, deg, b, w3, out):
    dinv, hlo, hhi = _halves(alo, ahi, ylo, yhi, deg, b[...])
    w3v = w3[...]
    t = (jnp.dot(hlo, w3v[:32], preferred_element_type=F32)
         + jnp.dot(hhi, w3v[32:], preferred_element_type=F32))
    y3 = t * dinv
    out[...] = jnp.concatenate([y3, jnp.zeros((BLK, 16 - OUT), F32)], axis=1)


def _tc_mm3(alo, ahi, ylo, yhi, deg, b, w3):
    return pl.pallas_call(
        _tc_mm3_body,
        grid=(N_BP // BLK,),
        in_specs=[_row_spec(32)] * 4 + [_row_spec(16), _full_spec((1, H)),
                                        _full_spec((H, OUT))],
        out_specs=_row_spec(16),
        out_shape=jax.ShapeDtypeStruct((N_BP, 16), F32),
    )(alo, ahi, ylo, yhi, deg, b.reshape(1, H), w3)


def _tc_final_body(p0, p1, y3, deg, b3, out):
    dinv = lax.rsqrt(deg[...][:, 0:1])
    x = dinv * (p0[...] + p1[...] + y3[...]) + b3[...]
    x2 = x[:, 0:OUT]
    m = jnp.max(x2, axis=1, keepdims=True)
    lse = m + jnp.log(jnp.sum(jnp.exp(x2 - m), axis=1, keepdims=True))
    out[...] = x2 - lse


def _tc_final(p0, p1, y3, deg_b, b3):
    b3p = jnp.concatenate([b3, jnp.zeros((16 - OUT,), F32)]).reshape(1, 16)
    return pl.pallas_call(
        _tc_final_body,
        grid=(N_BP // BLK,),
        in_specs=[_row_spec(16)] * 3 + [_row_spec(16), _full_spec((1, 16))],
        out_specs=_row_spec(OUT),
        out_shape=jax.ShapeDtypeStruct((N_B, OUT), F32),
    )(p0, p1, y3, deg_b, b3p)


def kernel(building_features, building_edge_index, community_features,
           community_edge_index, building_to_comm_mapping, W_c1, b_c1,
           W_c2, b_c2, W_att, b_att, W_b1, b_b1, W_b2, b_b2, W_b3, b_b3):
    bsrc, bdst = building_edge_index[0], building_edge_index[1]
    csrc, cdst = community_edge_index[0], community_edge_index[1]
    padb, padc = E_BP - E_B, E_CP - E_C
    bsrc2 = jnp.concatenate([bsrc, jnp.zeros((padb,), I32)]).reshape(-1, CH)
    bdst2 = jnp.concatenate([bdst, jnp.full((padb,), N_B, I32)]).reshape(-1, CH)
    csrc2 = jnp.concatenate([csrc, jnp.zeros((padc,), I32)]).reshape(-1, CH)
    cdst2 = jnp.concatenate([cdst, jnp.full((padc,), N_C, I32)]).reshape(-1, CH)
    map2 = jnp.concatenate(
        [building_to_comm_mapping,
         jnp.zeros((NMAP - N_B,), I32)]).reshape(-1, CH)

    deg_b, deg_c = _sc_degrees(bdst2, cdst2)

    y1lo, y1hi = _tc_comm1(community_features, W_c1, deg_c)
    a1lo, a1hi = _prop_comm(y1lo, y1hi, csrc2, cdst2)
    y2lo, y2hi = _tc_step(a1lo, a1hi, y1lo, y1hi, deg_c, W_c2, b_c1, N_CP)
    a2lo, a2hi = _prop_comm(y2lo, y2hi, csrc2, cdst2)
    comm_x = _tc_comm_fin(a2lo, a2hi, y2lo, y2hi, deg_c, b_c2)

    bc = _sc_gather_comm(comm_x, map2)
    yb1lo, yb1hi = _tc_att(building_features, bc, deg_b, W_att, b_att, W_b1)

    ab1lo, ab1hi = _prop_bldg(yb1lo, yb1hi, bsrc2, bdst2)
    yb2lo, yb2hi = _tc_step(ab1lo, ab1hi, yb1lo, yb1hi, deg_b, W_b2, b_b1,
                            N_BP)
    ab2lo, ab2hi = _prop_bldg(yb2lo, yb2hi, bsrc2, bdst2)
    y3 = _tc_mm3(ab2lo, ab2hi, yb2lo, yb2hi, deg_b, b_b2, W_b3)
    p0, p1 = _prop_out(y3, y3, bsrc2, bdst2)
    return _tc_final(p0, p1, y3, deg_b, b_b3)


# speedup vs baseline: 22.3736x; 1.2772x over previous
import functools

import jax
import jax.numpy as jnp
from jax import lax
from jax.experimental import pallas as pl
from jax.experimental.pallas import tpu as pltpu
from jax.experimental.pallas import tpu_sc as plsc

F32 = jnp.float32
I32 = jnp.int32

N_B, E_B, D_B = 50000, 800000, 64
N_C, E_C, D_C = 10000, 160000, 128
H, OUT = 64, 2

NCORE, NSUB = 2, 16
CH = 128
BLK = 512

N_BP = 50176
N_CP = 10240
E_BP = 6272 * CH
E_CP = 1280 * CH
NMAP = 416 * CH

_MESH = plsc.VectorSubcoreMesh(core_axis_name="c", subcore_axis_name="s",
                               num_cores=NCORE, num_subcores=NSUB)
_SC_PARAMS = pltpu.CompilerParams(use_tc_tiling_on_sc=False)


def _fill_const(buf, nrows, ncol16, val):
    def row(i, _):
        for k in range(ncol16):
            buf[i, pl.ds(k * 16, 16)] = jnp.full((16,), val, F32)
        return 0
    lax.fori_loop(0, nrows, row, 0)


_DEG_BSZ = 8


@functools.partial(
    pl.kernel,
    out_type=(jax.ShapeDtypeStruct((N_BP, 16), F32),
              jax.ShapeDtypeStruct((N_CP, 16), F32)),
    mesh=_MESH,
    scratch_types=[
        pltpu.VMEM((CH, 16), F32),
        pltpu.VMEM((3, _DEG_BSZ, CH), I32),
        pltpu.VMEM_SHARED((N_BP, 16), F32),
        pltpu.VMEM_SHARED((N_CP, 16), F32),
        pltpu.SemaphoreType.DMA,
        pltpu.SemaphoreType.DMA,
    ],
    compiler_params=_SC_PARAMS,
)
def _sc_degrees(bdst_ref, cdst_ref, degb_ref, degc_ref,
                ones_v, ebuf, accb, accc, isem, ssem):
    c = lax.axis_index("c")
    s = lax.axis_index("s")
    _fill_const(ones_v, CH, 1, 1.0)

    def init(acc, rows_per_tile):
        base = s * rows_per_tile
        nfull, rem = divmod(rows_per_tile, CH)

        def blk(t, _):
            pltpu.async_copy(ones_v, acc.at[pl.ds(base + t * CH, CH)], ssem)
            return 0
        lax.fori_loop(0, nfull, blk, 0)

        def blkw(t, _):
            pltpu.make_async_copy(
                ones_v, acc.at[pl.ds(base + t * CH, CH)], ssem).wait()
            return 0
        lax.fori_loop(0, nfull, blkw, 0)
        if rem:
            pltpu.sync_copy(ones_v.at[pl.ds(0, rem)],
                            acc.at[pl.ds(base + nfull * CH, rem)])

    def count(acc, dst_ref, nchunks):
        bsz = _DEG_BSZ
        per_tile = nchunks // NSUB
        nb = per_tile // bsz
        chunk0 = s * per_tile

        def load_idx(g):
            return pltpu.async_copy(
                dst_ref.at[pl.ds(chunk0 + g * bsz, bsz)],
                ebuf.at[lax.rem(g, 3)], isem)

        def wait_idx(g):
            pltpu.make_async_copy(dst_ref.at[pl.ds(chunk0, bsz)],
                                  ebuf.at[lax.rem(g, 3)], isem).wait()

        def fire_s(g):
            b3 = lax.rem(g, 3)
            for j in range(bsz):
                pltpu.async_copy(ones_v, acc.at[ebuf.at[b3, j]], ssem,
                                 add=True)

        def wait_s(g):
            b3 = lax.rem(g, 3)
            for j in range(bsz):
                pltpu.make_async_copy(ones_v, acc.at[ebuf.at[b3, j]],
                                      ssem).wait()

        pltpu.sync_copy(dst_ref.at[pl.ds(chunk0, bsz)], ebuf.at[0])
        load_idx(1)
        fire_s(0)

        def lbody(g, _):
            wait_idx(g)

            @pl.when(g >= 2)
            def _():
                wait_s(g - 2)
            fire_s(g)

            @pl.when(g + 1 < nb)
            def _():
                load_idx(g + 1)
            return 0
        lax.fori_loop(1, nb, lbody, 0)
        wait_s(nb - 2)
        wait_s(nb - 1)

    @pl.when(c == 0)
    def _():
        init(accb, N_BP // NSUB)

    @pl.when(c == 1)
    def _():
        init(accc, N_CP // NSUB)

    plsc.subcore_barrier()

    @pl.when(c == 0)
    def _():
        count(accb, bdst_ref, E_BP // CH)

    @pl.when(c == 1)
    def _():
        count(accc, cdst_ref, E_CP // CH)

    plsc.subcore_barrier()

    @pl.when(c == 0)
    def _():
        r = N_BP // NSUB
        pltpu.sync_copy(accb.at[pl.ds(s * r, r)], degb_ref.at[pl.ds(s * r, r)])

    @pl.when(c == 1)
    def _():
        r = N_CP // NSUB
        pltpu.sync_copy(accc.at[pl.ds(s * r, r)], degc_ref.at[pl.ds(s * r, r)])


def _make_propagate(nrows, nchunks, bsz, fdim, feature_split):
    per_tile = nchunks // NSUB if feature_split else nchunks // (NSUB * NCORE)
    nbatch, tail = divmod(per_tile, bsz)
    assert nbatch >= 3
    rpt = nrows // NSUB
    ZR = CH
    nfull, rem = divmod(rpt, ZR)

    def body(y0, y1, edges_ref, o0, o1, ebuf, rows, acc, isem, gsem, ssem):
        c = lax.axis_index("c")
        s = lax.axis_index("s")
        def zrow(i, _):
            for k in range(fdim // 16):
                rows[0, i, pl.ds(k * 16, 16)] = jnp.zeros((16,), F32)
            return 0
        lax.fori_loop(0, ZR, zrow, 0)
        base_r = s * rpt

        def zblk(t, _):
            pltpu.async_copy(rows.at[0, pl.ds(0, ZR)],
                             acc.at[pl.ds(base_r + t * ZR, ZR)], ssem)
            return 0
        lax.fori_loop(0, nfull, zblk, 0)

        def zblkw(t, _):
            pltpu.make_async_copy(rows.at[0, pl.ds(0, ZR)],
                                  acc.at[pl.ds(base_r + t * ZR, ZR)],
                                  ssem).wait()
            return 0
        lax.fori_loop(0, nfull, zblkw, 0)
        if rem:
            pltpu.sync_copy(rows.at[0, pl.ds(0, rem)],
                            acc.at[pl.ds(base_r + nfull * ZR, rem)])
        plsc.subcore_barrier()

        def run(tbl):
            if feature_split:
                chunk0 = s * per_tile
            else:
                chunk0 = c * (nchunks // NCORE) + s * per_tile

            def load_idx(g):
                pltpu.async_copy(edges_ref.at[pl.ds(chunk0 + g * bsz, bsz)],
                                 ebuf.at[lax.rem(g, 3)], isem)

            def wait_idx(g):
                pltpu.make_async_copy(edges_ref.at[pl.ds(chunk0, bsz)],
                                      ebuf.at[lax.rem(g, 3)], isem).wait()

            def fire_g(g):
                b3, b2 = lax.rem(g, 3), lax.rem(g, 2)
                for j in range(bsz):
                    pltpu.async_copy(tbl.at[ebuf.at[b3, j, 0]],
                                     rows.at[b2, pl.ds(j * CH, CH)], gsem)

            def wait_g(g):
                b3, b2 = lax.rem(g, 3), lax.rem(g, 2)
                for j in range(bsz):
                    pltpu.make_async_copy(
                        tbl.at[ebuf.at[b3, j, 0]],
                        rows.at[b2, pl.ds(j * CH, CH)], gsem).wait()

            def fire_s(g):
                b3, b2 = lax.rem(g, 3), lax.rem(g, 2)
                for j in range(bsz):
                    pltpu.async_copy(rows.at[b2, pl.ds(j * CH, CH)],
                                     acc.at[ebuf.at[b3, j, 1]], ssem,
                                     add=True)

            def wait_s(g):
                b3, b2 = lax.rem(g, 3), lax.rem(g, 2)
                for j in range(bsz):
                    pltpu.make_async_copy(
                        rows.at[b2, pl.ds(j * CH, CH)],
                        acc.at[ebuf.at[b3, j, 1]], ssem).wait()

            pltpu.sync_copy(edges_ref.at[pl.ds(chunk0, bsz)], ebuf.at[0])
            fire_g(0)
            load_idx(1)

            def lbody(g, _):
                wait_idx(g)

                @pl.when(g >= 2)
                def _():
                    wait_s(g - 2)
                fire_g(g)
                wait_g(g - 1)

                @pl.when(g + 1 < nbatch)
                def _():
                    load_idx(g + 1)
                fire_s(g - 1)
                return 0
            lax.fori_loop(1, nbatch, lbody, 0)
            wait_g(nbatch - 1)
            fire_s(nbatch - 1)
            wait_s(nbatch - 2)
            wait_s(nbatch - 1)
            for t in range(tail):
                cb = chunk0 + nbatch * bsz + t
                pltpu.sync_copy(edges_ref.at[pl.ds(cb, 1)],
                                ebuf.at[0, pl.ds(0, 1)])
                pltpu.sync_copy(tbl.at[ebuf.at[0, 0, 0]],
                                rows.at[0, pl.ds(0, CH)])
                pltpu.sync_copy(rows.at[0, pl.ds(0, CH)],
                                acc.at[ebuf.at[0, 0, 1]], add=True)

        if feature_split:
            @pl.when(c == 0)
            def _():
                run(y0)

            @pl.when(c == 1)
            def _():
                run(y1)
        else:
            run(y0)
        plsc.subcore_barrier()

        @pl.when(c == 0)
        def _():
            pltpu.sync_copy(acc.at[pl.ds(base_r, rpt)],
                            o0.at[pl.ds(base_r, rpt)])

        @pl.when(c == 1)
        def _():
            pltpu.sync_copy(acc.at[pl.ds(base_r, rpt)],
                            o1.at[pl.ds(base_r, rpt)])

    return pl.kernel(
        body,
        out_type=(jax.ShapeDtypeStruct((nrows, fdim), F32),
                  jax.ShapeDtypeStruct((nrows, fdim), F32)),
        mesh=_MESH,
        scratch_types=[
            pltpu.VMEM((3, bsz, 2, CH), I32),
            pltpu.VMEM((2, bsz * CH, fdim), F32),
            pltpu.VMEM_SHARED((nrows, fdim), F32),
            pltpu.SemaphoreType.DMA,
            pltpu.SemaphoreType.DMA,
            pltpu.SemaphoreType.DMA,
        ],
        compiler_params=_SC_PARAMS,
    )


_prop_comm = _make_propagate(N_CP, E_CP // CH, 8, 32, True)
_prop_bldg = _make_propagate(N_BP, E_BP // CH, 3, 32, True)
_prop_out = _make_propagate(N_BP, E_BP // CH, 8, 16, False)


@functools.partial(
    pl.kernel,
    out_type=jax.ShapeDtypeStruct((NMAP, D_B), F32),
    mesh=_MESH,
    scratch_types=[
        pltpu.VMEM((13, CH), I32),
        pltpu.VMEM((13 * CH, D_B), F32),
        pltpu.SemaphoreType.DMA,
    ],
    compiler_params=_SC_PARAMS,
)
def _sc_gather_comm(tbl_ref, map_ref, out_ref, idx_v, rows_v, sem):
    c = lax.axis_index("c")
    s = lax.axis_index("s")
    w = s * NCORE + c
    pltpu.sync_copy(map_ref.at[pl.ds(w * 13, 13)], idx_v)
    g = [pltpu.async_copy(tbl_ref.at[idx_v.at[j]],
                          rows_v.at[pl.ds(j * CH, CH)], sem)
         for j in range(13)]
    for d in g:
        d.wait()
    pltpu.sync_copy(rows_v, out_ref.at[pl.ds(w * 13 * CH, 13 * CH)])


def _row_spec(cols):
    return pl.BlockSpec((BLK, cols), lambda i: (i, 0))


def _full_spec(shape):
    return pl.BlockSpec(shape, lambda i: (0, 0))


def _tc_comm1_body(cf, w, deg, ylo, yhi):
    dinv = lax.rsqrt(deg[...][:, 0:1])
    y = jnp.dot(cf[...], w[...], preferred_element_type=F32) * dinv
    ylo[...] = y[:, :32]
    yhi[...] = y[:, 32:]


def _tc_comm1(cf, w_c1, deg_c):
    return pl.pallas_call(
        _tc_comm1_body,
        grid=(N_CP // BLK,),
        in_specs=[_row_spec(D_C), _full_spec((D_C, H)), _row_spec(16)],
        out_specs=[_row_spec(32), _row_spec(32)],
        out_shape=[jax.ShapeDtypeStruct((N_CP, 32), F32)] * 2,
    )(cf, w_c1, deg_c)


def _halves(alo, ahi, ylo, yhi, deg, bv):
    dinv = lax.rsqrt(deg[...][:, 0:1])
    hlo = jnp.maximum(dinv * (alo[...] + ylo[...]) + bv[:, :32], 0.0)
    hhi = jnp.maximum(dinv * (ahi[...] + yhi[...]) + bv[:, 32:], 0.0)
    return dinv, hlo, hhi


def _tc_step_body(alo, ahi, ylo, yhi, deg, w, b, olo, ohi):
    dinv, hlo, hhi = _halves(alo, ahi, ylo, yhi, deg, b[...])
    wv = w[...]
    t = (jnp.dot(hlo, wv[:32], preferred_element_type=F32)
         + jnp.dot(hhi, wv[32:], preferred_element_type=F32))
    y = t * dinv
    olo[...] = y[:, :32]
    ohi[...] = y[:, 32:]


def _tc_step(alo, ahi, ylo, yhi, deg, w, b, npad):
    return pl.pallas_call(
        _tc_step_body,
        grid=(npad // BLK,),
        in_specs=[_row_spec(32)] * 4 + [_row_spec(16), _full_spec((H, H)),
                                        _full_spec((1, H))],
        out_specs=[_row_spec(32), _row_spec(32)],
        out_shape=[jax.ShapeDtypeStruct((npad, 32), F32)] * 2,
    )(alo, ahi, ylo, yhi, deg, w, b.reshape(1, H))


def _tc_comm_fin_body(alo, ahi, ylo, yhi, deg, b, out):
    _, hlo, hhi = _halves(alo, ahi, ylo, yhi, deg, b[...])
    out[...] = jnp.concatenate([hlo, hhi], axis=1)


def _tc_comm_fin(alo, ahi, ylo, yhi, deg, b):
    return pl.pallas_call(
        _tc_comm_fin_body,
        grid=(N_CP // BLK,),
        in_specs=[_row_spec(32)] * 4 + [_row_spec(16), _full_spec((1, H))],
        out_specs=_row_spec(H),
        out_shape=jax.ShapeDtypeStruct((N_C, H), F32),
    )(alo, ahi, ylo, yhi, deg, b.reshape(1, H))


def _tc_att_body(bf, bc, deg, wa, w1, ba, olo, ohi):
    dinv = lax.rsqrt(deg[...][:, 0:1])
    bfv, bcv = bf[...], bc[...]
    wav, w1v = wa[...], w1[...]
    z = (jnp.dot(bfv, wav[:D_B], preferred_element_type=F32)
         + jnp.dot(bcv, wav[D_B:], preferred_element_type=F32) + ba[...])
    m = jnp.max(z, axis=1, keepdims=True)
    e = jnp.exp(z - m)
    a = e / jnp.sum(e, axis=1, keepdims=True)
    t = (a[:, 0:1] * jnp.dot(bfv, w1v[:D_B], preferred_element_type=F32)
         + a[:, 1:2] * jnp.dot(bcv, w1v[D_B:], preferred_element_type=F32))
    y = t * dinv
    olo[...] = y[:, :32]
    ohi[...] = y[:, 32:]


def _tc_att(bf, bc, deg_b, w_att, b_att, w_b1):
    return pl.pallas_call(
        _tc_att_body,
        grid=(N_BP // BLK,),
        in_specs=[_row_spec(D_B), _row_spec(D_B), _row_spec(16),
                  _full_spec((D_B + H, OUT)), _full_spec((D_B + H, H)),
                  _full_spec((1, OUT))],
        out_specs=[_row_spec(32), _row_spec(32)],
        out_shape=[jax.ShapeDtypeStruct((N_BP, 32), F32)] * 2,
    )(bf, bc, deg_b, w_att, w_b1, b_att.reshape(1, OUT))


def _tc_mm3_body(alo, ahi, ylo, yhi, deg, b, w3, out):
    dinv, hlo, hhi = _halves(alo, ahi, ylo, yhi, deg, b[...])
    w3v = w3[...]
    t = (jnp.dot(hlo, w3v[:32], preferred_element_type=F32)
         + jnp.dot(hhi, w3v[32:], preferred_element_type=F32))
    y3 = t * dinv
    out[...] = jnp.concatenate([y3, jnp.zeros((BLK, 16 - OUT), F32)], axis=1)


def _tc_mm3(alo, ahi, ylo, yhi, deg, b, w3):
    return pl.pallas_call(
        _tc_mm3_body,
        grid=(N_BP // BLK,),
        in_specs=[_row_spec(32)] * 4 + [_row_spec(16), _full_spec((1, H)),
                                        _full_spec((H, OUT))],
        out_specs=_row_spec(16),
        out_shape=jax.ShapeDtypeStruct((N_BP, 16), F32),
    )(alo, ahi, ylo, yhi, deg, b.reshape(1, H), w3)


def _tc_final_body(p0, p1, y3, deg, b3, out):
    dinv = lax.rsqrt(deg[...][:, 0:1])
    x = dinv * (p0[...] + p1[...] + y3[...]) + b3[...]
    x2 = x[:, 0:OUT]
    m = jnp.max(x2, axis=1, keepdims=True)
    lse = m + jnp.log(jnp.sum(jnp.exp(x2 - m), axis=1, keepdims=True))
    out[...] = x2 - lse


def _tc_final(p0, p1, y3, deg_b, b3):
    b3p = jnp.concatenate([b3, jnp.zeros((16 - OUT,), F32)]).reshape(1, 16)
    return pl.pallas_call(
        _tc_final_body,
        grid=(N_BP // BLK,),
        in_specs=[_row_spec(16)] * 3 + [_row_spec(16), _full_spec((1, 16))],
        out_specs=_row_spec(OUT),
        out_shape=jax.ShapeDtypeStruct((N_B, OUT), F32),
    )(p0, p1, y3, deg_b, b3p)


def kernel(building_features, building_edge_index, community_features,
           community_edge_index, building_to_comm_mapping, W_c1, b_c1,
           W_c2, b_c2, W_att, b_att, W_b1, b_b1, W_b2, b_b2, W_b3, b_b3):
    bsrc, bdst = building_edge_index[0], building_edge_index[1]
    csrc, cdst = community_edge_index[0], community_edge_index[1]
    padb, padc = E_BP - E_B, E_CP - E_C
    bsrc2 = jnp.concatenate([bsrc, jnp.zeros((padb,), I32)]).reshape(-1, CH)
    bdst2 = jnp.concatenate([bdst, jnp.full((padb,), N_B, I32)]).reshape(-1, CH)
    csrc2 = jnp.concatenate([csrc, jnp.zeros((padc,), I32)]).reshape(-1, CH)
    cdst2 = jnp.concatenate([cdst, jnp.full((padc,), N_C, I32)]).reshape(-1, CH)
    bedges = jnp.stack([bsrc2, bdst2], axis=1)
    cedges = jnp.stack([csrc2, cdst2], axis=1)
    map2 = jnp.concatenate(
        [building_to_comm_mapping,
         jnp.zeros((NMAP - N_B,), I32)]).reshape(-1, CH)

    deg_b, deg_c = _sc_degrees(bdst2, cdst2)

    y1lo, y1hi = _tc_comm1(community_features, W_c1, deg_c)
    a1lo, a1hi = _prop_comm(y1lo, y1hi, cedges)
    y2lo, y2hi = _tc_step(a1lo, a1hi, y1lo, y1hi, deg_c, W_c2, b_c1, N_CP)
    a2lo, a2hi = _prop_comm(y2lo, y2hi, cedges)
    comm_x = _tc_comm_fin(a2lo, a2hi, y2lo, y2hi, deg_c, b_c2)

    bc = _sc_gather_comm(comm_x, map2)
    yb1lo, yb1hi = _tc_att(building_features, bc, deg_b, W_att, b_att, W_b1)

    ab1lo, ab1hi = _prop_bldg(yb1lo, yb1hi, bedges)
    yb2lo, yb2hi = _tc_step(ab1lo, ab1hi, yb1lo, yb1hi, deg_b, W_b2, b_b1,
                            N_BP)
    ab2lo, ab2hi = _prop_bldg(yb2lo, yb2hi, bedges)
    y3 = _tc_mm3(ab2lo, ab2hi, yb2lo, yb2hi, deg_b, b_b2, W_b3)
    p0, p1 = _prop_out(y3, y3, bedges)
    return _tc_final(p0, p1, y3, deg_b, b_b3)


# speedup vs baseline: 26.0304x; 1.1634x over previous
import functools

import jax
import jax.numpy as jnp
from jax import lax
from jax.experimental import pallas as pl
from jax.experimental.pallas import tpu as pltpu
from jax.experimental.pallas import tpu_sc as plsc

F32 = jnp.float32
I32 = jnp.int32

N_B, E_B, D_B = 50000, 800000, 64
N_C, E_C, D_C = 10000, 160000, 128
H, OUT = 64, 2

NCORE, NSUB = 2, 16
CH = 128
BLK = 512

N_BP = 50176
N_CP = 10240
E_BP = 6272 * CH
E_CP = 1280 * CH
NMAP = 392 * CH

_MESH = plsc.VectorSubcoreMesh(core_axis_name="c", subcore_axis_name="s",
                               num_cores=NCORE, num_subcores=NSUB)
_SC_PARAMS = pltpu.CompilerParams(use_tc_tiling_on_sc=False)


def _fill_const(buf, nrows, ncol16, val):
    def row(i, _):
        for k in range(ncol16):
            buf[i, pl.ds(k * 16, 16)] = jnp.full((16,), val, F32)
        return 0
    lax.fori_loop(0, nrows, row, 0)


_DEG_BSZ = 8


@functools.partial(
    pl.kernel,
    out_type=(jax.ShapeDtypeStruct((N_BP, 128), F32),
              jax.ShapeDtypeStruct((N_CP, 128), F32)),
    mesh=_MESH,
    scratch_types=[
        pltpu.VMEM((CH, 16), F32),
        pltpu.VMEM((3, _DEG_BSZ, CH), I32),
        pltpu.VMEM_SHARED((N_BP, 16), F32),
        pltpu.VMEM_SHARED((N_CP, 16), F32),
        pltpu.SemaphoreType.DMA,
        pltpu.SemaphoreType.DMA,
    ],
    compiler_params=_SC_PARAMS,
)
def _sc_degrees(bdst_ref, cdst_ref, degb_ref, degc_ref,
                ones_v, ebuf, accb, accc, isem, ssem):
    c = lax.axis_index("c")
    s = lax.axis_index("s")
    _fill_const(ones_v, CH, 1, 1.0)

    def init(acc, rows_per_tile):
        base = s * rows_per_tile
        nfull, rem = divmod(rows_per_tile, CH)

        def blk(t, _):
            pltpu.async_copy(ones_v, acc.at[pl.ds(base + t * CH, CH)], ssem)
            return 0
        lax.fori_loop(0, nfull, blk, 0)

        def blkw(t, _):
            pltpu.make_async_copy(
                ones_v, acc.at[pl.ds(base + t * CH, CH)], ssem).wait()
            return 0
        lax.fori_loop(0, nfull, blkw, 0)
        if rem:
            pltpu.sync_copy(ones_v.at[pl.ds(0, rem)],
                            acc.at[pl.ds(base + nfull * CH, rem)])

    def count(acc, dst_ref, nchunks):
        bsz = _DEG_BSZ
        per_tile = nchunks // NSUB
        nb = per_tile // bsz
        chunk0 = s * per_tile

        def load_idx(g):
            return pltpu.async_copy(
                dst_ref.at[pl.ds(chunk0 + g * bsz, bsz)],
                ebuf.at[lax.rem(g, 3)], isem)

        def wait_idx(g):
            pltpu.make_async_copy(dst_ref.at[pl.ds(chunk0, bsz)],
                                  ebuf.at[lax.rem(g, 3)], isem).wait()

        def fire_s(g):
            b3 = lax.rem(g, 3)
            for j in range(bsz):
                pltpu.async_copy(ones_v, acc.at[ebuf.at[b3, j]], ssem,
                                 add=True)

        def wait_s(g):
            b3 = lax.rem(g, 3)
            for j in range(bsz):
                pltpu.make_async_copy(ones_v, acc.at[ebuf.at[b3, j]],
                                      ssem).wait()

        pltpu.sync_copy(dst_ref.at[pl.ds(chunk0, bsz)], ebuf.at[0])
        load_idx(1)
        fire_s(0)

        def lbody(g, _):
            wait_idx(g)

            @pl.when(g >= 2)
            def _():
                wait_s(g - 2)
            fire_s(g)

            @pl.when(g + 1 < nb)
            def _():
                load_idx(g + 1)
            return 0
        lax.fori_loop(1, nb, lbody, 0)
        wait_s(nb - 2)
        wait_s(nb - 1)

    @pl.when(c == 0)
    def _():
        init(accb, N_BP // NSUB)

    @pl.when(c == 1)
    def _():
        init(accc, N_CP // NSUB)

    plsc.subcore_barrier()

    @pl.when(c == 0)
    def _():
        count(accb, bdst_ref, E_BP // CH)

    @pl.when(c == 1)
    def _():
        count(accc, cdst_ref, E_CP // CH)

    plsc.subcore_barrier()

    @pl.when(c == 0)
    def _():
        r = N_BP // NSUB
        pltpu.sync_copy(accb.at[pl.ds(s * r, r)],
                        degb_ref.at[pl.ds(s * r, r), pl.ds(0, 16)])

    @pl.when(c == 1)
    def _():
        r = N_CP // NSUB
        pltpu.sync_copy(accc.at[pl.ds(s * r, r)],
                        degc_ref.at[pl.ds(s * r, r), pl.ds(0, 16)])


def _make_propagate(nrows, nchunks, bsz, fdim, feature_split):
    per_tile = nchunks // NSUB if feature_split else nchunks // (NSUB * NCORE)
    nbatch, tail = divmod(per_tile, bsz)
    assert nbatch >= 3
    rpt = nrows // NSUB
    ZR = CH
    nfull, rem = divmod(rpt, ZR)

    def body(y0, y1, edges_ref, out_o, ebuf, rows, acc, isem, gsem, ssem):
        c = lax.axis_index("c")
        s = lax.axis_index("s")
        base_r = s * rpt

        def init_from(tbl):
            pltpu.sync_copy(tbl.at[pl.ds(base_r, rpt)],
                            acc.at[pl.ds(base_r, rpt)])

        def init_zero():
            def zrow(i, _):
                for k in range(fdim // 16):
                    rows[0, i, pl.ds(k * 16, 16)] = jnp.zeros((16,), F32)
                return 0
            lax.fori_loop(0, ZR, zrow, 0)

            def zblk(t, _):
                pltpu.async_copy(rows.at[0, pl.ds(0, ZR)],
                                 acc.at[pl.ds(base_r + t * ZR, ZR)], ssem)
                return 0
            lax.fori_loop(0, nfull, zblk, 0)

            def zblkw(t, _):
                pltpu.make_async_copy(rows.at[0, pl.ds(0, ZR)],
                                      acc.at[pl.ds(base_r + t * ZR, ZR)],
                                      ssem).wait()
                return 0
            lax.fori_loop(0, nfull, zblkw, 0)
            if rem:
                pltpu.sync_copy(rows.at[0, pl.ds(0, rem)],
                                acc.at[pl.ds(base_r + nfull * ZR, rem)])

        if feature_split:
            @pl.when(c == 0)
            def _():
                init_from(y0)

            @pl.when(c == 1)
            def _():
                init_from(y1)
        else:
            @pl.when(c == 0)
            def _():
                init_from(y0)

            @pl.when(c == 1)
            def _():
                init_zero()
        plsc.subcore_barrier()

        def run(tbl):
            if feature_split:
                chunk0 = s * per_tile
            else:
                chunk0 = c * (nchunks // NCORE) + s * per_tile

            def load_idx(g):
                pltpu.async_copy(edges_ref.at[pl.ds(chunk0 + g * bsz, bsz)],
                                 ebuf.at[lax.rem(g, 3)], isem)

            def wait_idx(g):
                pltpu.make_async_copy(edges_ref.at[pl.ds(chunk0, bsz)],
                                      ebuf.at[lax.rem(g, 3)], isem).wait()

            def fire_g(g):
                b3, b2 = lax.rem(g, 3), lax.rem(g, 2)
                for j in range(bsz):
                    pltpu.async_copy(tbl.at[ebuf.at[b3, j, 0]],
                                     rows.at[b2, pl.ds(j * CH, CH)], gsem)

            def wait_g(g):
                b3, b2 = lax.rem(g, 3), lax.rem(g, 2)
                for j in range(bsz):
                    pltpu.make_async_copy(
                        tbl.at[ebuf.at[b3, j, 0]],
                        rows.at[b2, pl.ds(j * CH, CH)], gsem).wait()

            def fire_s(g):
                b3, b2 = lax.rem(g, 3), lax.rem(g, 2)
                for j in range(bsz):
                    pltpu.async_copy(rows.at[b2, pl.ds(j * CH, CH)],
                                     acc.at[ebuf.at[b3, j, 1]], ssem,
                                     add=True)

            def wait_s(g):
                b3, b2 = lax.rem(g, 3), lax.rem(g, 2)
                for j in range(bsz):
                    pltpu.make_async_copy(
                        rows.at[b2, pl.ds(j * CH, CH)],
                        acc.at[ebuf.at[b3, j, 1]], ssem).wait()

            pltpu.sync_copy(edges_ref.at[pl.ds(chunk0, bsz)], ebuf.at[0])
            fire_g(0)
            load_idx(1)

            def lbody(g, _):
                wait_idx(g)

                @pl.when(g >= 2)
                def _():
                    wait_s(g - 2)
                fire_g(g)
                wait_g(g - 1)

                @pl.when(g + 1 < nbatch)
                def _():
                    load_idx(g + 1)
                fire_s(g - 1)
                return 0
            lax.fori_loop(1, nbatch, lbody, 0)
            wait_g(nbatch - 1)
            fire_s(nbatch - 1)
            wait_s(nbatch - 2)
            wait_s(nbatch - 1)
            for t in range(tail):
                cb = chunk0 + nbatch * bsz + t
                pltpu.sync_copy(edges_ref.at[pl.ds(cb, 1)],
                                ebuf.at[0, pl.ds(0, 1)])
                pltpu.sync_copy(tbl.at[ebuf.at[0, 0, 0]],
                                rows.at[0, pl.ds(0, CH)])
                pltpu.sync_copy(rows.at[0, pl.ds(0, CH)],
                                acc.at[ebuf.at[0, 0, 1]], add=True)

        if feature_split:
            @pl.when(c == 0)
            def _():
                run(y0)

            @pl.when(c == 1)
            def _():
                run(y1)
        else:
            run(y0)
        plsc.subcore_barrier()

        pltpu.sync_copy(acc.at[pl.ds(base_r, rpt)],
                        out_o.at[pl.ds(base_r, rpt), pl.ds(fdim * c, fdim)])

    return pl.kernel(
        body,
        out_type=jax.ShapeDtypeStruct((nrows, 128), F32),
        mesh=_MESH,
        scratch_types=[
            pltpu.VMEM((3, bsz, 2, CH), I32),
            pltpu.VMEM((2, bsz * CH, fdim), F32),
            pltpu.VMEM_SHARED((nrows, fdim), F32),
            pltpu.SemaphoreType.DMA,
            pltpu.SemaphoreType.DMA,
            pltpu.SemaphoreType.DMA,
        ],
        compiler_params=_SC_PARAMS,
    )


_prop_comm = _make_propagate(N_CP, E_CP // CH, 8, 32, True)
_prop_bldg = _make_propagate(N_BP, E_BP // CH, 3, 32, True)
_prop_out = _make_propagate(N_BP, E_BP // CH, 8, 16, False)


@functools.partial(
    pl.kernel,
    out_type=jax.ShapeDtypeStruct((NMAP, 128), F32),
    mesh=_MESH,
    scratch_types=[
        pltpu.VMEM((13, CH), I32),
        pltpu.VMEM((13 * CH, D_B), F32),
        pltpu.SemaphoreType.DMA,
    ],
    compiler_params=_SC_PARAMS,
)
def _sc_gather_comm(tbl_ref, map_ref, out_ref, idx_v, rows_v, sem):
    c = lax.axis_index("c")
    s = lax.axis_index("s")
    w = s * NCORE + c
    base = w * 12 + jnp.minimum(w, 8)

    def run(nch):
        pltpu.sync_copy(map_ref.at[pl.ds(base, nch)],
                        idx_v.at[pl.ds(0, nch)])
        g = [pltpu.async_copy(tbl_ref.at[idx_v.at[j]],
                              rows_v.at[pl.ds(j * CH, CH)], sem)
             for j in range(nch)]
        for d in g:
            d.wait()
        pltpu.sync_copy(rows_v.at[pl.ds(0, nch * CH)],
                        out_ref.at[pl.ds(base * CH, nch * CH),
                                   pl.ds(0, D_B)])

    @pl.when(w < 8)
    def _():
        run(13)

    @pl.when(w >= 8)
    def _():
        run(12)


def _row_spec(cols):
    return pl.BlockSpec((BLK, cols), lambda i: (i, 0))


def _full_spec(shape):
    return pl.BlockSpec(shape, lambda i: (0, 0))


def _tc_comm1_body(cf, w, deg, ylo, yhi):
    dinv = lax.rsqrt(deg[...][:, 0:1])
    y = jnp.dot(cf[...], w[...], preferred_element_type=F32) * dinv
    ylo[...] = y[:, :32]
    yhi[...] = y[:, 32:]


def _tc_comm1(cf, w_c1, degw_c):
    return pl.pallas_call(
        _tc_comm1_body,
        grid=(N_CP // BLK,),
        in_specs=[_row_spec(D_C), _full_spec((D_C, H)), _row_spec(128)],
        out_specs=[_row_spec(32), _row_spec(32)],
        out_shape=[jax.ShapeDtypeStruct((N_CP, 32), F32)] * 2,
    )(cf, w_c1, degw_c)


def _tc_step_body(o, deg, w, b, ylo, yhi):
    dinv = lax.rsqrt(deg[...][:, 0:1])
    h = jnp.maximum(dinv * o[...][:, 0:H] + b[...], 0.0)
    y = jnp.dot(h, w[...], preferred_element_type=F32) * dinv
    ylo[...] = y[:, :32]
    yhi[...] = y[:, 32:]


def _tc_step(o, degw, w, b, npad):
    return pl.pallas_call(
        _tc_step_body,
        grid=(npad // BLK,),
        in_specs=[_row_spec(128), _row_spec(128), _full_spec((H, H)),
                  _full_spec((1, H))],
        out_specs=[_row_spec(32), _row_spec(32)],
        out_shape=[jax.ShapeDtypeStruct((npad, 32), F32)] * 2,
    )(o, degw, w, b.reshape(1, H))


def _tc_comm_fin_body(o, deg, b, out):
    dinv = lax.rsqrt(deg[...][:, 0:1])
    out[...] = jnp.maximum(dinv * o[...][:, 0:H] + b[...], 0.0)


def _tc_comm_fin(o, degw_c, b):
    return pl.pallas_call(
        _tc_comm_fin_body,
        grid=(N_CP // BLK,),
        in_specs=[_row_spec(128), _row_spec(128), _full_spec((1, H))],
        out_specs=_row_spec(H),
        out_shape=jax.ShapeDtypeStruct((N_C, H), F32),
    )(o, degw_c, b.reshape(1, H))


def _tc_att_body(bf, bc, deg, wa, w1, ba, olo, ohi):
    dinv = lax.rsqrt(deg[...][:, 0:1])
    bfv, bcv = bf[...], bc[...][:, 0:D_B]
    wav, w1v = wa[...], w1[...]
    z = (jnp.dot(bfv, wav[:D_B], preferred_element_type=F32)
         + jnp.dot(bcv, wav[D_B:], preferred_element_type=F32) + ba[...])
    m = jnp.max(z, axis=1, keepdims=True)
    e = jnp.exp(z - m)
    a = e / jnp.sum(e, axis=1, keepdims=True)
    t = (a[:, 0:1] * jnp.dot(bfv, w1v[:D_B], preferred_element_type=F32)
         + a[:, 1:2] * jnp.dot(bcv, w1v[D_B:], preferred_element_type=F32))
    y = t * dinv
    olo[...] = y[:, :32]
    ohi[...] = y[:, 32:]


def _tc_att(bf, bc, degw_b, w_att, b_att, w_b1):
    return pl.pallas_call(
        _tc_att_body,
        grid=(N_BP // BLK,),
        in_specs=[_row_spec(D_B), _row_spec(128), _row_spec(128),
                  _full_spec((D_B + H, OUT)), _full_spec((D_B + H, H)),
                  _full_spec((1, OUT))],
        out_specs=[_row_spec(32), _row_spec(32)],
        out_shape=[jax.ShapeDtypeStruct((N_BP, 32), F32)] * 2,
    )(bf, bc, degw_b, w_att, w_b1, b_att.reshape(1, OUT))


def _tc_mm3_body(o, deg, b, w3, out):
    dinv = lax.rsqrt(deg[...][:, 0:1])
    h = jnp.maximum(dinv * o[...][:, 0:H] + b[...], 0.0)
    y3 = jnp.dot(h, w3[...], preferred_element_type=F32) * dinv
    out[...] = jnp.concatenate([y3, jnp.zeros((BLK, 16 - OUT), F32)], axis=1)


def _tc_mm3(o, degw_b, b, w3):
    return pl.pallas_call(
        _tc_mm3_body,
        grid=(N_BP // BLK,),
        in_specs=[_row_spec(128), _row_spec(128), _full_spec((1, H)),
                  _full_spec((H, OUT))],
        out_specs=_row_spec(16),
        out_shape=jax.ShapeDtypeStruct((N_BP, 16), F32),
    )(o, degw_b, b.reshape(1, H), w3)


def _tc_final_body(o3, deg, b3, out):
    dinv = lax.rsqrt(deg[...][:, 0:1])
    ov = o3[...]
    x = dinv * (ov[:, 0:16] + ov[:, 16:32]) + b3[...]
    x2 = x[:, 0:OUT]
    m = jnp.max(x2, axis=1, keepdims=True)
    lse = m + jnp.log(jnp.sum(jnp.exp(x2 - m), axis=1, keepdims=True))
    out[...] = x2 - lse


def _tc_final(o3, degw_b, b3):
    b3p = jnp.concatenate([b3, jnp.zeros((16 - OUT,), F32)]).reshape(1, 16)
    return pl.pallas_call(
        _tc_final_body,
        grid=(N_BP // BLK,),
        in_specs=[_row_spec(128), _row_spec(128), _full_spec((1, 16))],
        out_specs=_row_spec(OUT),
        out_shape=jax.ShapeDtypeStruct((N_B, OUT), F32),
    )(o3, degw_b, b3p)


def kernel(building_features, building_edge_index, community_features,
           community_edge_index, building_to_comm_mapping, W_c1, b_c1,
           W_c2, b_c2, W_att, b_att, W_b1, b_b1, W_b2, b_b2, W_b3, b_b3):
    bsrc, bdst = building_edge_index[0], building_edge_index[1]
    csrc, cdst = community_edge_index[0], community_edge_index[1]
    padb, padc = E_BP - E_B, E_CP - E_C
    bsrc2 = jnp.concatenate([bsrc, jnp.zeros((padb,), I32)]).reshape(-1, CH)
    bdst2 = jnp.concatenate([bdst, jnp.full((padb,), N_B, I32)]).reshape(-1, CH)
    csrc2 = jnp.concatenate([csrc, jnp.zeros((padc,), I32)]).reshape(-1, CH)
    cdst2 = jnp.concatenate([cdst, jnp.full((padc,), N_C, I32)]).reshape(-1, CH)
    bedges = jnp.stack([bsrc2, bdst2], axis=1)
    cedges = jnp.stack([csrc2, cdst2], axis=1)
    map2 = jnp.concatenate(
        [building_to_comm_mapping,
         jnp.zeros((NMAP - N_B,), I32)]).reshape(-1, CH)

    degw_b, degw_c = _sc_degrees(bdst2, cdst2)

    y1lo, y1hi = _tc_comm1(community_features, W_c1, degw_c)
    o_c1 = _prop_comm(y1lo, y1hi, cedges)
    y2lo, y2hi = _tc_step(o_c1, degw_c, W_c2, b_c1, N_CP)
    o_c2 = _prop_comm(y2lo, y2hi, cedges)
    comm_x = _tc_comm_fin(o_c2, degw_c, b_c2)

    bc = _sc_gather_comm(comm_x, map2)
    yb1lo, yb1hi = _tc_att(building_features, bc, degw_b, W_att, b_att, W_b1)

    o_b1 = _prop_bldg(yb1lo, yb1hi, bedges)
    yb2lo, yb2hi = _tc_step(o_b1, degw_b, W_b2, b_b1, N_BP)
    o_b2 = _prop_bldg(yb2lo, yb2hi, bedges)
    y3 = _tc_mm3(o_b2, degw_b, b_b2, W_b3)
    o3 = _prop_out(y3, y3, bedges)
    return _tc_final(o3, degw_b, b_b3)


# speedup vs baseline: 26.7475x; 1.0275x over previous
import functools

import jax
import jax.numpy as jnp
from jax import lax
from jax.experimental import pallas as pl
from jax.experimental.pallas import tpu as pltpu
from jax.experimental.pallas import tpu_sc as plsc

F32 = jnp.float32
I32 = jnp.int32

N_B, E_B, D_B = 50000, 800000, 64
N_C, E_C, D_C = 10000, 160000, 128
H, OUT = 64, 2

NCORE, NSUB = 2, 16
CH = 128
BLK = 512

N_BP = 50176
N_CP = 10240
E_BP = 6272 * CH
E_CP = 1280 * CH
NMAP = 392 * CH

_MESH = plsc.VectorSubcoreMesh(core_axis_name="c", subcore_axis_name="s",
                               num_cores=NCORE, num_subcores=NSUB)
_SC_PARAMS = pltpu.CompilerParams(use_tc_tiling_on_sc=False)


def _fill_const(buf, nrows, ncol16, val):
    def row(i, _):
        for k in range(ncol16):
            buf[i, pl.ds(k * 16, 16)] = jnp.full((16,), val, F32)
        return 0
    lax.fori_loop(0, nrows, row, 0)


_DEG_BSZ = 8


@functools.partial(
    pl.kernel,
    out_type=(jax.ShapeDtypeStruct((N_BP, 128), F32),
              jax.ShapeDtypeStruct((N_CP, 128), F32),
              jax.ShapeDtypeStruct((N_CP, 16), F32)),
    mesh=_MESH,
    scratch_types=[
        pltpu.VMEM((CH, 16), F32),
        pltpu.VMEM((3, _DEG_BSZ, CH), I32),
        pltpu.VMEM_SHARED((N_BP, 16), F32),
        pltpu.VMEM_SHARED((N_CP, 16), F32),
        pltpu.SemaphoreType.DMA,
        pltpu.SemaphoreType.DMA,
    ],
    compiler_params=_SC_PARAMS,
)
def _sc_degrees(bdst_ref, cdst_ref, degb_ref, degc_ref, degc16_ref,
                ones_v, ebuf, accb, accc, isem, ssem):
    c = lax.axis_index("c")
    s = lax.axis_index("s")
    _fill_const(ones_v, CH, 1, 1.0)

    def init(acc, rows_per_tile):
        base = s * rows_per_tile
        nfull, rem = divmod(rows_per_tile, CH)

        def blk(t, _):
            pltpu.async_copy(ones_v, acc.at[pl.ds(base + t * CH, CH)], ssem)
            return 0
        lax.fori_loop(0, nfull, blk, 0)

        def blkw(t, _):
            pltpu.make_async_copy(
                ones_v, acc.at[pl.ds(base + t * CH, CH)], ssem).wait()
            return 0
        lax.fori_loop(0, nfull, blkw, 0)
        if rem:
            pltpu.sync_copy(ones_v.at[pl.ds(0, rem)],
                            acc.at[pl.ds(base + nfull * CH, rem)])

    def count(acc, dst_ref, nchunks):
        bsz = _DEG_BSZ
        per_tile = nchunks // NSUB
        nb = per_tile // bsz
        chunk0 = s * per_tile

        def load_idx(g):
            return pltpu.async_copy(
                dst_ref.at[pl.ds(chunk0 + g * bsz, bsz)],
                ebuf.at[lax.rem(g, 3)], isem)

        def wait_idx(g):
            pltpu.make_async_copy(dst_ref.at[pl.ds(chunk0, bsz)],
                                  ebuf.at[lax.rem(g, 3)], isem).wait()

        def fire_s(g):
            b3 = lax.rem(g, 3)
            for j in range(bsz):
                pltpu.async_copy(ones_v, acc.at[ebuf.at[b3, j]], ssem,
                                 add=True)

        def wait_s(g):
            b3 = lax.rem(g, 3)
            for j in range(bsz):
                pltpu.make_async_copy(ones_v, acc.at[ebuf.at[b3, j]],
                                      ssem).wait()

        pltpu.sync_copy(dst_ref.at[pl.ds(chunk0, bsz)], ebuf.at[0])
        load_idx(1)
        fire_s(0)

        def lbody(g, _):
            wait_idx(g)

            @pl.when(g >= 2)
            def _():
                wait_s(g - 2)
            fire_s(g)

            @pl.when(g + 1 < nb)
            def _():
                load_idx(g + 1)
            return 0
        lax.fori_loop(1, nb, lbody, 0)
        wait_s(nb - 2)
        wait_s(nb - 1)

    @pl.when(c == 0)
    def _():
        init(accb, N_BP // NSUB)

    @pl.when(c == 1)
    def _():
        init(accc, N_CP // NSUB)

    plsc.subcore_barrier()

    @pl.when(c == 0)
    def _():
        count(accb, bdst_ref, E_BP // CH)

    @pl.when(c == 1)
    def _():
        count(accc, cdst_ref, E_CP // CH)

    plsc.subcore_barrier()

    @pl.when(c == 0)
    def _():
        r = N_BP // NSUB
        pltpu.sync_copy(accb.at[pl.ds(s * r, r)],
                        degb_ref.at[pl.ds(s * r, r), pl.ds(0, 16)])

    @pl.when(c == 1)
    def _():
        r = N_CP // NSUB
        pltpu.sync_copy(accc.at[pl.ds(s * r, r)],
                        degc_ref.at[pl.ds(s * r, r), pl.ds(0, 16)])
        pltpu.sync_copy(accc.at[pl.ds(s * r, r)],
                        degc16_ref.at[pl.ds(s * r, r)])


def _make_propagate(nrows, nchunks, bsz, fdim, feature_split,
                    with_gather=False):
    per_tile = nchunks // NSUB if feature_split else nchunks // (NSUB * NCORE)
    nbatch, tail = divmod(per_tile, bsz)
    assert nbatch >= 3
    rpt = nrows // NSUB
    ZR = CH
    nfull, rem = divmod(rpt, ZR)

    def body(y0, y1, edges_ref, *rest):
        if with_gather:
            (map_ref, degc_ref, out_o, out_g, tcomp,
             ebuf, rows, acc, mbuf, dbuf, isem, gsem, ssem) = rest
        else:
            out_o, ebuf, rows, acc, isem, gsem, ssem = rest
        c = lax.axis_index("c")
        s = lax.axis_index("s")
        base_r = s * rpt

        def init_from(tbl):
            pltpu.sync_copy(tbl.at[pl.ds(base_r, rpt)],
                            acc.at[pl.ds(base_r, rpt)])

        def init_zero():
            def zrow(i, _):
                for k in range(fdim // 16):
                    rows[0, i, pl.ds(k * 16, 16)] = jnp.zeros((16,), F32)
                return 0
            lax.fori_loop(0, ZR, zrow, 0)

            def zblk(t, _):
                pltpu.async_copy(rows.at[0, pl.ds(0, ZR)],
                                 acc.at[pl.ds(base_r + t * ZR, ZR)], ssem)
                return 0
            lax.fori_loop(0, nfull, zblk, 0)

            def zblkw(t, _):
                pltpu.make_async_copy(rows.at[0, pl.ds(0, ZR)],
                                      acc.at[pl.ds(base_r + t * ZR, ZR)],
                                      ssem).wait()
                return 0
            lax.fori_loop(0, nfull, zblkw, 0)
            if rem:
                pltpu.sync_copy(rows.at[0, pl.ds(0, rem)],
                                acc.at[pl.ds(base_r + nfull * ZR, rem)])

        if feature_split:
            @pl.when(c == 0)
            def _():
                init_from(y0)

            @pl.when(c == 1)
            def _():
                init_from(y1)
        else:
            @pl.when(c == 0)
            def _():
                init_from(y0)

            @pl.when(c == 1)
            def _():
                init_zero()
        plsc.subcore_barrier()

        def run(tbl):
            if feature_split:
                chunk0 = s * per_tile
            else:
                chunk0 = c * (nchunks // NCORE) + s * per_tile

            def load_idx(g):
                pltpu.async_copy(edges_ref.at[pl.ds(chunk0 + g * bsz, bsz)],
                                 ebuf.at[lax.rem(g, 3)], isem)

            def wait_idx(g):
                pltpu.make_async_copy(edges_ref.at[pl.ds(chunk0, bsz)],
                                      ebuf.at[lax.rem(g, 3)], isem).wait()

            def fire_g(g):
                b3, b2 = lax.rem(g, 3), lax.rem(g, 2)
                for j in range(bsz):
                    pltpu.async_copy(tbl.at[ebuf.at[b3, j, 0]],
                                     rows.at[b2, pl.ds(j * CH, CH)], gsem)

            def wait_g(g):
                b3, b2 = lax.rem(g, 3), lax.rem(g, 2)
                for j in range(bsz):
                    pltpu.make_async_copy(
                        tbl.at[ebuf.at[b3, j, 0]],
                        rows.at[b2, pl.ds(j * CH, CH)], gsem).wait()

            def fire_s(g):
                b3, b2 = lax.rem(g, 3), lax.rem(g, 2)
                for j in range(bsz):
                    pltpu.async_copy(rows.at[b2, pl.ds(j * CH, CH)],
                                     acc.at[ebuf.at[b3, j, 1]], ssem,
                                     add=True)

            def wait_s(g):
                b3, b2 = lax.rem(g, 3), lax.rem(g, 2)
                for j in range(bsz):
                    pltpu.make_async_copy(
                        rows.at[b2, pl.ds(j * CH, CH)],
                        acc.at[ebuf.at[b3, j, 1]], ssem).wait()

            pltpu.sync_copy(edges_ref.at[pl.ds(chunk0, bsz)], ebuf.at[0])
            fire_g(0)
            load_idx(1)

            def lbody(g, _):
                wait_idx(g)

                @pl.when(g >= 2)
                def _():
                    wait_s(g - 2)
                fire_g(g)
                wait_g(g - 1)

                @pl.when(g + 1 < nbatch)
                def _():
                    load_idx(g + 1)
                fire_s(g - 1)
                return 0
            lax.fori_loop(1, nbatch, lbody, 0)
            wait_g(nbatch - 1)
            fire_s(nbatch - 1)
            wait_s(nbatch - 2)
            wait_s(nbatch - 1)
            for t in range(tail):
                cb = chunk0 + nbatch * bsz + t
                pltpu.sync_copy(edges_ref.at[pl.ds(cb, 1)],
                                ebuf.at[0, pl.ds(0, 1)])
                pltpu.sync_copy(tbl.at[ebuf.at[0, 0, 0]],
                                rows.at[0, pl.ds(0, CH)])
                pltpu.sync_copy(rows.at[0, pl.ds(0, CH)],
                                acc.at[ebuf.at[0, 0, 1]], add=True)

        if feature_split:
            @pl.when(c == 0)
            def _():
                run(y0)

            @pl.when(c == 1)
            def _():
                run(y1)
        else:
            run(y0)
        plsc.subcore_barrier()

        if with_gather:
            pltpu.sync_copy(acc.at[pl.ds(base_r, rpt)],
                            tcomp.at[c, pl.ds(base_r, rpt)])
            plsc.subcore_barrier()
        else:
            pltpu.sync_copy(acc.at[pl.ds(base_r, rpt)],
                            out_o.at[pl.ds(base_r, rpt), pl.ds(fdim * c, fdim)])

        if with_gather:
            mbase = s * 24 + jnp.minimum(s, 8)

            def gather_run(nch):
                pltpu.sync_copy(map_ref.at[pl.ds(mbase, nch)],
                                mbuf.at[pl.ds(0, nch)])
                for w0 in range(0, nch, 8):
                    n = min(8, nch - w0)
                    g = [pltpu.async_copy(
                            tcomp.at[c].at[mbuf.at[w0 + j]],
                            rows.at[0, pl.ds(j * CH, CH)], gsem)
                         for j in range(n)]
                    d = [pltpu.async_copy(
                            degc_ref.at[mbuf.at[w0 + j]],
                            dbuf.at[pl.ds(j * CH, CH)], gsem)
                         for j in range(n)]
                    for x in g + d:
                        x.wait()
                    r0 = (mbase + w0) * CH
                    pltpu.sync_copy(
                        rows.at[0, pl.ds(0, n * CH)],
                        out_g.at[pl.ds(r0, n * CH), pl.ds(fdim * c, fdim)])
                    pltpu.sync_copy(
                        dbuf.at[pl.ds(0, n * CH)],
                        out_g.at[pl.ds(r0, n * CH), pl.ds(64 + 16 * c, 16)])

            @pl.when(s < 8)
            def _():
                gather_run(25)

            @pl.when(s >= 8)
            def _():
                gather_run(24)

    out_type = jax.ShapeDtypeStruct((nrows, 128), F32)
    scratch = [
        pltpu.VMEM((3, bsz, 2, CH), I32),
        pltpu.VMEM((2, bsz * CH, fdim), F32),
        pltpu.VMEM_SHARED((nrows, fdim), F32),
    ]
    sems = [pltpu.SemaphoreType.DMA] * 3
    if with_gather:
        out_type = (out_type, jax.ShapeDtypeStruct((NMAP, 128), F32),
                    jax.ShapeDtypeStruct((NCORE, nrows, fdim), F32))
        scratch += [pltpu.VMEM((25, CH), I32),
                    pltpu.VMEM((8 * CH, 16), F32)]
    return pl.kernel(
        body,
        out_type=out_type,
        mesh=_MESH,
        scratch_types=scratch + sems,
        compiler_params=_SC_PARAMS,
    )


_prop_comm = _make_propagate(N_CP, E_CP // CH, 8, 32, True)
_prop_comm_g = _make_propagate(N_CP, E_CP // CH, 8, 32, True, with_gather=True)
_prop_bldg = _make_propagate(N_BP, E_BP // CH, 3, 32, True)
_prop_out = _make_propagate(N_BP, E_BP // CH, 8, 16, False)


def _row_spec(cols):
    return pl.BlockSpec((BLK, cols), lambda i: (i, 0))


def _full_spec(shape):
    return pl.BlockSpec(shape, lambda i: (0, 0))


def _tc_comm1_body(cf, w, deg, ylo, yhi):
    dinv = lax.rsqrt(deg[...][:, 0:1])
    y = jnp.dot(cf[...], w[...], preferred_element_type=F32) * dinv
    ylo[...] = y[:, :32]
    yhi[...] = y[:, 32:]


def _tc_comm1(cf, w_c1, degw_c):
    return pl.pallas_call(
        _tc_comm1_body,
        grid=(N_CP // BLK,),
        in_specs=[_row_spec(D_C), _full_spec((D_C, H)), _row_spec(128)],
        out_specs=[_row_spec(32), _row_spec(32)],
        out_shape=[jax.ShapeDtypeStruct((N_CP, 32), F32)] * 2,
    )(cf, w_c1, degw_c)


def _tc_step_body(o, deg, w, b, ylo, yhi):
    dinv = lax.rsqrt(deg[...][:, 0:1])
    h = jnp.maximum(dinv * o[...][:, 0:H] + b[...], 0.0)
    y = jnp.dot(h, w[...], preferred_element_type=F32) * dinv
    ylo[...] = y[:, :32]
    yhi[...] = y[:, 32:]


def _tc_step(o, degw, w, b, npad):
    return pl.pallas_call(
        _tc_step_body,
        grid=(npad // BLK,),
        in_specs=[_row_spec(128), _row_spec(128), _full_spec((H, H)),
                  _full_spec((1, H))],
        out_specs=[_row_spec(32), _row_spec(32)],
        out_shape=[jax.ShapeDtypeStruct((npad, 32), F32)] * 2,
    )(o, degw, w, b.reshape(1, H))


def _tc_att_body(bf, g, deg, wa, w1, ba, bc2, olo, ohi):
    dinv = lax.rsqrt(deg[...][:, 0:1])
    gv = g[...]
    dinvc = lax.rsqrt(gv[:, 64:65])
    bcv = jnp.maximum(dinvc * gv[:, 0:D_B] + bc2[...], 0.0)
    bfv = bf[...]
    wav, w1v = wa[...], w1[...]
    z = (jnp.dot(bfv, wav[:D_B], preferred_element_type=F32)
         + jnp.dot(bcv, wav[D_B:], preferred_element_type=F32) + ba[...])
    m = jnp.max(z, axis=1, keepdims=True)
    e = jnp.exp(z - m)
    a = e / jnp.sum(e, axis=1, keepdims=True)
    t = (a[:, 0:1] * jnp.dot(bfv, w1v[:D_B], preferred_element_type=F32)
         + a[:, 1:2] * jnp.dot(bcv, w1v[D_B:], preferred_element_type=F32))
    y = t * dinv
    olo[...] = y[:, :32]
    ohi[...] = y[:, 32:]


def _tc_att(bf, g, degw_b, w_att, b_att, w_b1, b_c2):
    return pl.pallas_call(
        _tc_att_body,
        grid=(N_BP // BLK,),
        in_specs=[_row_spec(D_B), _row_spec(128), _row_spec(128),
                  _full_spec((D_B + H, OUT)), _full_spec((D_B + H, H)),
                  _full_spec((1, OUT)), _full_spec((1, H))],
        out_specs=[_row_spec(32), _row_spec(32)],
        out_shape=[jax.ShapeDtypeStruct((N_BP, 32), F32)] * 2,
    )(bf, g, degw_b, w_att, w_b1, b_att.reshape(1, OUT), b_c2.reshape(1, H))


def _tc_mm3_body(o, deg, b, w3, out):
    dinv = lax.rsqrt(deg[...][:, 0:1])
    h = jnp.maximum(dinv * o[...][:, 0:H] + b[...], 0.0)
    y3 = jnp.dot(h, w3[...], preferred_element_type=F32) * dinv
    out[...] = jnp.concatenate([y3, jnp.zeros((BLK, 16 - OUT), F32)], axis=1)


def _tc_mm3(o, degw_b, b, w3):
    return pl.pallas_call(
        _tc_mm3_body,
        grid=(N_BP // BLK,),
        in_specs=[_row_spec(128), _row_spec(128), _full_spec((1, H)),
                  _full_spec((H, OUT))],
        out_specs=_row_spec(16),
        out_shape=jax.ShapeDtypeStruct((N_BP, 16), F32),
    )(o, degw_b, b.reshape(1, H), w3)


def _tc_final_body(o3, deg, b3, out):
    dinv = lax.rsqrt(deg[...][:, 0:1])
    ov = o3[...]
    x = dinv * (ov[:, 0:16] + ov[:, 16:32]) + b3[...]
    x2 = x[:, 0:OUT]
    m = jnp.max(x2, axis=1, keepdims=True)
    lse = m + jnp.log(jnp.sum(jnp.exp(x2 - m), axis=1, keepdims=True))
    out[...] = x2 - lse


def _tc_final(o3, degw_b, b3):
    b3p = jnp.concatenate([b3, jnp.zeros((16 - OUT,), F32)]).reshape(1, 16)
    return pl.pallas_call(
        _tc_final_body,
        grid=(N_BP // BLK,),
        in_specs=[_row_spec(128), _row_spec(128), _full_spec((1, 16))],
        out_specs=_row_spec(OUT),
        out_shape=jax.ShapeDtypeStruct((N_B, OUT), F32),
    )(o3, degw_b, b3p)


def kernel(building_features, building_edge_index, community_features,
           community_edge_index, building_to_comm_mapping, W_c1, b_c1,
           W_c2, b_c2, W_att, b_att, W_b1, b_b1, W_b2, b_b2, W_b3, b_b3):
    bsrc, bdst = building_edge_index[0], building_edge_index[1]
    csrc, cdst = community_edge_index[0], community_edge_index[1]
    padb, padc = E_BP - E_B, E_CP - E_C
    bsrc2 = jnp.concatenate([bsrc, jnp.zeros((padb,), I32)]).reshape(-1, CH)
    bdst2 = jnp.concatenate([bdst, jnp.full((padb,), N_B, I32)]).reshape(-1, CH)
    csrc2 = jnp.concatenate([csrc, jnp.zeros((padc,), I32)]).reshape(-1, CH)
    cdst2 = jnp.concatenate([cdst, jnp.full((padc,), N_C, I32)]).reshape(-1, CH)
    bedges = jnp.stack([bsrc2, bdst2], axis=1)
    cedges = jnp.stack([csrc2, cdst2], axis=1)
    map2 = jnp.concatenate(
        [building_to_comm_mapping,
         jnp.zeros((NMAP - N_B,), I32)]).reshape(-1, CH)

    degw_b, degw_c, degc16 = _sc_degrees(bdst2, cdst2)

    y1lo, y1hi = _tc_comm1(community_features, W_c1, degw_c)
    o_c1 = _prop_comm(y1lo, y1hi, cedges)
    y2lo, y2hi = _tc_step(o_c1, degw_c, W_c2, b_c1, N_CP)
    _, g, _ = _prop_comm_g(y2lo, y2hi, cedges, map2, degc16)

    yb1lo, yb1hi = _tc_att(building_features, g, degw_b, W_att, b_att, W_b1,
                           b_c2)

    o_b1 = _prop_bldg(yb1lo, yb1hi, bedges)
    yb2lo, yb2hi = _tc_step(o_b1, degw_b, W_b2, b_b1, N_BP)
    o_b2 = _prop_bldg(yb2lo, yb2hi, bedges)
    y3 = _tc_mm3(o_b2, degw_b, b_b2, W_b3)
    o3 = _prop_out(y3, y3, bedges)
    return _tc_final(o3, degw_b, b_b3)
